# baseline (speedup 1.0000x reference)
import functools
import jax
import jax.numpy as jnp
from jax import lax
from jax.experimental import pallas as pl
from jax.experimental.pallas import tpu as pltpu
from jax.experimental.pallas import tpu_sc as plsc

NQ = 1000
ND = 10000
NG = 11000
D = 128
NQ_P = 1008
ND_P = 10008
NG_P = 11008

NC = 2
NS = 16
NW = NC * NS
K = 128

CQ = 4
CD = 80
CG = 44
NB = 4

_MESH = plsc.VectorSubcoreMesh(core_axis_name="c", subcore_axis_name="s")
_HIGH = jax.lax.Precision.HIGHEST


def _dot(a, b):
    return jnp.dot(a, b, precision=_HIGH, preferred_element_type=jnp.float32)


def _segsum_body(qtab, dtab, qsrc, qdst, dsrc, ddst, z2d,
                 outq, outd, accq, accd, srcv, dstv, rows, sem):
    c = lax.axis_index("c")
    s = lax.axis_index("s")
    wid = s * NC + c

    @pl.when(s == 0)
    def _zero():
        pltpu.sync_copy(z2d.at[pl.ds(0, NQ_P)], accq)
        pltpu.sync_copy(z2d.at[pl.ds(0, ND_P)], accd)

    plsc.subcore_barrier()

    def pipe(tab, acc, src_h, dst_h, base, n_chunks):
        def body(i, _):
            b = base + i * K
            pltpu.sync_copy(src_h.at[pl.ds(b, K)], srcv)
            pltpu.sync_copy(dst_h.at[pl.ds(b, K)], dstv)
            pltpu.async_copy(tab.at[srcv], rows, sem).wait()
            pltpu.sync_copy(rows, acc.at[dstv], add=True)
            return _
        lax.fori_loop(0, n_chunks, body, None)

    pipe(qtab, accq, qsrc, qdst, wid * (CQ * K), CQ)
    pipe(dtab, accd, dsrc, ddst, wid * (CD * K), CD)

    plsc.subcore_barrier()

    @pl.when(s == 0)
    def _dump():
        pltpu.sync_copy(accq, outq.at[c])
        pltpu.sync_copy(accd, outd.at[c])


_segsum_call = pl.kernel(
    _segsum_body,
    out_type=(
        jax.ShapeDtypeStruct((NC, NQ_P, D), jnp.float32),
        jax.ShapeDtypeStruct((NC, ND_P, D), jnp.float32),
    ),
    mesh=_MESH,
    scratch_types=[
        pltpu.VMEM_SHARED((NQ_P, D), jnp.float32),
        pltpu.VMEM_SHARED((ND_P, D), jnp.float32),
        pltpu.VMEM((K,), jnp.int32),
        pltpu.VMEM((K,), jnp.int32),
        pltpu.VMEM((K, D), jnp.float32),
        pltpu.SemaphoreType.DMA,
    ],
)


def _gat_body(hg, gsrc, gdst, es_h, ed_h, m_h, z2d, z1d,
              outp, outden, acc, den_sh, srcv, dstv, esb, edb, wv, rows,
              mv, sem):
    c = lax.axis_index("c")
    s = lax.axis_index("s")
    wid = s * NC + c

    @pl.when(s == 0)
    def _zero():
        pltpu.sync_copy(z2d.at[pl.ds(0, NG_P)], acc)
        pltpu.sync_copy(z1d.at[pl.ds(0, NG_P)], den_sh)

    pltpu.sync_copy(m_h, mv)

    plsc.subcore_barrier()

    mvec = mv[...]
    base = wid * (CG * K)

    def chunk(i, _):
        b = base + i * K
        pltpu.sync_copy(gsrc.at[pl.ds(b, K)], srcv)
        pltpu.sync_copy(gdst.at[pl.ds(b, K)], dstv)
        pltpu.async_copy(es_h.at[srcv], esb, sem).wait()
        pltpu.async_copy(ed_h.at[dstv], edb, sem).wait()
        for t in range(K // 16):
            e16 = esb[pl.ds(t * 16, 16)] + edb[pl.ds(t * 16, 16)]
            e16 = jnp.maximum(e16, 0.2 * e16)
            wv[pl.ds(t * 16, 16)] = jnp.exp(e16 - mvec)
        pltpu.sync_copy(wv, den_sh.at[dstv], add=True)
        pltpu.async_copy(hg.at[srcv], rows, sem).wait()

        def scale(t, _):
            w16 = wv[pl.ds(t * 16, 16)]
            for j in range(16):
                wj = jnp.full((16,), w16[j], jnp.float32)
                r = t * 16 + j
                for g in range(D // 16):
                    rows[r, pl.ds(g * 16, 16)] = rows[r, pl.ds(g * 16, 16)] * wj
            return _

        lax.fori_loop(0, K // 16, scale, None)
        pltpu.sync_copy(rows, acc.at[dstv], add=True)
        return _

    lax.fori_loop(0, CG, chunk, None)

    plsc.subcore_barrier()

    @pl.when(s == 0)
    def _dump():
        pltpu.sync_copy(acc, outp.at[c])
        pltpu.sync_copy(den_sh, outden.at[c])


_gat_call = pl.kernel(
    _gat_body,
    out_type=(
        jax.ShapeDtypeStruct((NC, NG_P, D), jnp.float32),
        jax.ShapeDtypeStruct((NC, NG_P), jnp.float32),
    ),
    mesh=_MESH,
    scratch_types=[
        pltpu.VMEM_SHARED((NG_P, D), jnp.float32),
        pltpu.VMEM_SHARED((NG_P,), jnp.float32),
        pltpu.VMEM((K,), jnp.int32),
        pltpu.VMEM((K,), jnp.int32),
        pltpu.VMEM((K,), jnp.float32),
        pltpu.VMEM((K,), jnp.float32),
        pltpu.VMEM((K,), jnp.float32),
        pltpu.VMEM((K, D), jnp.float32),
        pltpu.VMEM((16,), jnp.float32),
        pltpu.SemaphoreType.DMA,
    ],
)


def _tc_data_gin_body(dx, sd1p, dW1, db1, dW2, pD):
    agg = dx[...] + sd1p[0] + sd1p[1]
    h = jnp.maximum(_dot(agg, dW1[...]) + db1[...][None, :], 0.0)
    pD[...] = _dot(h, dW2[...])


def _tc_query_gat_body(qx, dx, sq1p, qW1, qb1, qW2, gW, ga_src, ga_dst,
                       pQ, hg, esq, esd, edq, edd, mrow):
    aggq = qx[...] + sq1p[0, :NQ, :] + sq1p[1, :NQ, :]
    hq = jnp.maximum(_dot(aggq, qW1[...]) + qb1[...][None, :], 0.0)
    pQ[...] = _dot(hq, qW2[...])
    hgq = _dot(qx[...], gW[...])
    hgd = _dot(dx[...], gW[...])
    hg[0:NQ, :] = hgq
    hg[NQ:NG, :] = hgd
    hg[NG:NG_P, :] = jnp.zeros((NG_P - NG, D), jnp.float32)
    a_s = ga_src[...][None, :]
    a_d = ga_dst[...][None, :]
    vq_s = jnp.sum(hgq * a_s, axis=1)
    vd_s = jnp.sum(hgd * a_s, axis=1)
    vq_d = jnp.sum(hgq * a_d, axis=1)
    vd_d = jnp.sum(hgd * a_d, axis=1)
    esq[...] = vq_s
    esd[...] = vd_s
    edq[...] = vq_d
    edd[...] = vd_d
    mes = jnp.maximum(jnp.maximum(jnp.max(vq_s), jnp.max(vd_s)), 0.0)
    med = jnp.maximum(jnp.maximum(jnp.max(vq_d), jnp.max(vd_d)), 0.0)
    mb = mes + med
    m = jnp.maximum(mb, 0.2 * mb)
    mrow[...] = jnp.full((1, D), m, jnp.float32)


def _make_node_out_body(n):
    def body(p, sp, b, out, pool):
        x = p[...] + sp[0, :n, :] + sp[1, :n, :] + b[...][None, :]
        out[...] = x
        pool[...] = jnp.sum(x, axis=0, keepdims=True)
    return body


def _tc_gat_pool_body(outp_p, outden_p, qpool2, dpool2):
    op = outp_p[0] + outp_p[1]
    den = (outden_p[0] + outden_p[1]).reshape(1, NG_P)
    r = 1.0 / (den + 1e-16)
    cols = lax.broadcasted_iota(jnp.int32, (1, NG_P), 1)
    rq = jnp.where(cols < NQ, r, 0.0)
    rd = jnp.where((cols >= NQ) & (cols < NG), r, 0.0)
    qpool2[...] = _dot(rq, op)
    dpool2[...] = _dot(rd, op)


def _tc_mlp_body(qpool, qpool2, dpool, dpool2,
                 l1W, l1b, l2W, l2b, l3W, l3b, l4W, l4b, out):
    pooled = jnp.concatenate(
        [qpool[...], qpool2[...], dpool[...], dpool2[...]], axis=1)
    h = _dot(pooled, l1W[...]) + l1b[...][None, :]
    h = _dot(h, l2W[...]) + l2b[...][None, :]
    h = jnp.maximum(h, 0.0)
    h = _dot(h, l3W[...]) + l3b[...][None, :]
    h = jnp.maximum(h, 0.0)
    h = _dot(h, l4W[...]) + l4b[...][None, :]
    out[...] = jnp.maximum(h, 0.0)


def _pad_edges(edges, n_chunks_per_worker, dummy):
    e = edges.astype(jnp.int32)
    total = NW * n_chunks_per_worker * K
    pad = total - e.shape[1]
    src = jnp.concatenate([e[0], jnp.zeros((pad,), jnp.int32)])
    dst = jnp.concatenate([e[1], jnp.full((pad,), dummy, jnp.int32)])
    return src, dst


def kernel(query_in_feat, data_in_feat, query_edge_list, data_edge_list,
           query2data_edge_list, qW1, qb1, qW2, qb2, dW1, db1, dW2, db2,
           gW, ga_src, ga_dst, l1W, l1b, l2W, l2b, l3W, l3b, l4W, l4b):
    qx = query_in_feat
    dx = data_in_feat
    qsrc, qdst = _pad_edges(query_edge_list, CQ, NQ)
    dsrc, ddst = _pad_edges(data_edge_list, CD, ND)
    gsrc, gdst = _pad_edges(query2data_edge_list, CG, NG)
    z2d = jnp.zeros((NG_P, D), jnp.float32)
    z1d = jnp.zeros((NG_P,), jnp.float32)

    sq1p, sd1p = _segsum_call(qx, dx, qsrc, qdst, dsrc, ddst, z2d)

    pD = pl.pallas_call(
        _tc_data_gin_body,
        grid=(10,),
        in_specs=[
            pl.BlockSpec((1000, D), lambda i: (i, 0)),
            pl.BlockSpec((NC, 1000, D), lambda i: (0, i, 0)),
            pl.BlockSpec((D, 256), lambda i: (0, 0)),
            pl.BlockSpec((256,), lambda i: (0,)),
            pl.BlockSpec((256, D), lambda i: (0, 0)),
        ],
        out_specs=pl.BlockSpec((1000, D), lambda i: (i, 0)),
        out_shape=jax.ShapeDtypeStruct((ND, D), jnp.float32),
    )(dx, sd1p, dW1, db1, dW2)

    pQ, hg, esq, esd, edq, edd, mrow = pl.pallas_call(
        _tc_query_gat_body,
        out_shape=(
            jax.ShapeDtypeStruct((NQ, D), jnp.float32),
            jax.ShapeDtypeStruct((NG_P, D), jnp.float32),
            jax.ShapeDtypeStruct((NQ,), jnp.float32),
            jax.ShapeDtypeStruct((ND,), jnp.float32),
            jax.ShapeDtypeStruct((NQ,), jnp.float32),
            jax.ShapeDtypeStruct((ND,), jnp.float32),
            jax.ShapeDtypeStruct((1, D), jnp.float32),
        ),
    )(qx, dx, sq1p, qW1, qb1, qW2, gW, ga_src, ga_dst)

    zpad = jnp.zeros((NG_P - NG,), jnp.float32)
    es = jnp.concatenate([esq, esd, zpad])
    ed = jnp.concatenate([edq, edd, zpad])
    m16 = mrow[0, :16]

    sq2p, sd2p = _segsum_call(pQ, pD, qsrc, qdst, dsrc, ddst, z2d)

    outp_p, outden_p = _gat_call(hg, gsrc, gdst, es, ed, m16, z2d, z1d)

    query_x, qpool = pl.pallas_call(
        _make_node_out_body(NQ),
        out_shape=(
            jax.ShapeDtypeStruct((NQ, D), jnp.float32),
            jax.ShapeDtypeStruct((1, D), jnp.float32),
        ),
    )(pQ, sq2p, qb2)
    data_x, dpool = pl.pallas_call(
        _make_node_out_body(ND),
        out_shape=(
            jax.ShapeDtypeStruct((ND, D), jnp.float32),
            jax.ShapeDtypeStruct((1, D), jnp.float32),
        ),
    )(pD, sd2p, db2)
    qpool2, dpool2 = pl.pallas_call(
        _tc_gat_pool_body,
        out_shape=(
            jax.ShapeDtypeStruct((1, D), jnp.float32),
            jax.ShapeDtypeStruct((1, D), jnp.float32),
        ),
    )(outp_p, outden_p)

    h4 = pl.pallas_call(
        _tc_mlp_body,
        out_shape=jax.ShapeDtypeStruct((1, 8 * 4674), jnp.float32),
    )(qpool, qpool2, dpool, dpool2, l1W, l1b, l2W, l2b, l3W, l3b, l4W, l4b)

    pred = h4.reshape(8, 4674)
    return (pred, query_x, data_x)


# speedup vs baseline: 1.5158x; 1.5158x over previous
import functools
import jax
import jax.numpy as jnp
from jax import lax
from jax.experimental import pallas as pl
from jax.experimental.pallas import tpu as pltpu
from jax.experimental.pallas import tpu_sc as plsc

NQ = 1000
ND = 10000
NG = 11000
D = 128
NQ_P = 1008
ND_P = 10008
NG_P = 11008

NC = 2
NS = 16
NW = NC * NS
K = 128

CQ = 4
CD = 79
CG = 43

_MESH = plsc.VectorSubcoreMesh(core_axis_name="c", subcore_axis_name="s")
_HIGH = jax.lax.Precision.HIGHEST


def _dot(a, b):
    return jnp.dot(a, b, precision=_HIGH, preferred_element_type=jnp.float32)


def _segsum_body(qtab, dtab, qsrc, qdst, dsrc, ddst, z2d,
                 outq, outd, accq, accd, srcv, dstv, rows, sem):
    c = lax.axis_index("c")
    s = lax.axis_index("s")
    wid = s * NC + c

    @pl.when(s == 0)
    def _zero():
        pltpu.sync_copy(z2d.at[pl.ds(0, NQ_P)], accq)
        pltpu.sync_copy(z2d.at[pl.ds(0, ND_P)], accd)

    plsc.subcore_barrier()

    def pipe(tab, acc, src_h, dst_h, base, n_chunks):
        def body(i, _):
            b = base + i * K
            pltpu.sync_copy(src_h.at[pl.ds(b, K)], srcv)
            pltpu.sync_copy(dst_h.at[pl.ds(b, K)], dstv)
            pltpu.async_copy(tab.at[srcv], rows, sem).wait()
            pltpu.sync_copy(rows, acc.at[dstv], add=True)
            return _
        lax.fori_loop(0, n_chunks, body, None)

    pipe(qtab, accq, qsrc, qdst, wid * (CQ * K), CQ)
    pipe(dtab, accd, dsrc, ddst, wid * (CD * K), CD)

    plsc.subcore_barrier()

    @pl.when(s == 0)
    def _dump():
        pltpu.sync_copy(accq, outq.at[c])
        pltpu.sync_copy(accd, outd.at[c])


_segsum_call = pl.kernel(
    _segsum_body,
    out_type=(
        jax.ShapeDtypeStruct((NC, NQ_P, D), jnp.float32),
        jax.ShapeDtypeStruct((NC, ND_P, D), jnp.float32),
    ),
    mesh=_MESH,
    scratch_types=[
        pltpu.VMEM_SHARED((NQ_P, D), jnp.float32),
        pltpu.VMEM_SHARED((ND_P, D), jnp.float32),
        pltpu.VMEM((K,), jnp.int32),
        pltpu.VMEM((K,), jnp.int32),
        pltpu.VMEM((K, D), jnp.float32),
        pltpu.SemaphoreType.DMA,
    ],
)


def _gat_body(hg, gsrc, gdst, es_h, ed_h, m_h, z2d, z1d,
              outp, outden, acc, den_sh, srcv, dstv, esb, edb, wv, rows,
              mv, sem):
    c = lax.axis_index("c")
    s = lax.axis_index("s")
    wid = s * NC + c

    @pl.when(s == 0)
    def _zero():
        pltpu.sync_copy(z2d.at[pl.ds(0, NG_P)], acc)
        pltpu.sync_copy(z1d.at[pl.ds(0, NG_P)], den_sh)

    pltpu.sync_copy(m_h, mv)

    plsc.subcore_barrier()

    mvec = mv[...]
    base = wid * (CG * K)

    def chunk(i, _):
        b = base + i * K
        pltpu.sync_copy(gsrc.at[pl.ds(b, K)], srcv)
        pltpu.sync_copy(gdst.at[pl.ds(b, K)], dstv)
        pltpu.async_copy(es_h.at[srcv], esb, sem).wait()
        pltpu.async_copy(ed_h.at[dstv], edb, sem).wait()
        for t in range(K // 16):
            e16 = esb[pl.ds(t * 16, 16)] + edb[pl.ds(t * 16, 16)]
            e16 = jnp.maximum(e16, 0.2 * e16)
            wv[pl.ds(t * 16, 16)] = jnp.exp(e16 - mvec)
        pltpu.sync_copy(wv, den_sh.at[dstv], add=True)
        pltpu.async_copy(hg.at[srcv], rows, sem).wait()

        def scale(t, _):
            w16 = wv[pl.ds(t * 16, 16)]
            for j in range(16):
                wj = jnp.full((16,), w16[j], jnp.float32)
                r = t * 16 + j
                for g in range(D // 16):
                    rows[r, pl.ds(g * 16, 16)] = rows[r, pl.ds(g * 16, 16)] * wj
            return _

        lax.fori_loop(0, K // 16, scale, None)
        pltpu.sync_copy(rows, acc.at[dstv], add=True)
        return _

    lax.fori_loop(0, CG, chunk, None)

    plsc.subcore_barrier()

    @pl.when(s == 0)
    def _dump():
        pltpu.sync_copy(acc, outp.at[c])
        pltpu.sync_copy(den_sh, outden.at[c])


_gat_call = pl.kernel(
    _gat_body,
    out_type=(
        jax.ShapeDtypeStruct((NC, NG_P, D), jnp.float32),
        jax.ShapeDtypeStruct((NC, NG_P), jnp.float32),
    ),
    mesh=_MESH,
    scratch_types=[
        pltpu.VMEM_SHARED((NG_P, D), jnp.float32),
        pltpu.VMEM_SHARED((NG_P,), jnp.float32),
        pltpu.VMEM((K,), jnp.int32),
        pltpu.VMEM((K,), jnp.int32),
        pltpu.VMEM((K,), jnp.float32),
        pltpu.VMEM((K,), jnp.float32),
        pltpu.VMEM((K,), jnp.float32),
        pltpu.VMEM((K, D), jnp.float32),
        pltpu.VMEM((16,), jnp.float32),
        pltpu.SemaphoreType.DMA,
    ],
)


def _tc_data_gin_body(dx, sd1p, dW1, db1, dW2, pD):
    agg = dx[...] + sd1p[0] + sd1p[1]
    h = jnp.maximum(_dot(agg, dW1[...]) + db1[...][None, :], 0.0)
    pD[...] = _dot(h, dW2[...])


def _tc_query_gat_body(qx, dx, sq1p, qW1, qb1, qW2, gW, ga_src, ga_dst,
                       pQ, hg, esq, esd, edq, edd, mrow):
    aggq = qx[...] + sq1p[0, :NQ, :] + sq1p[1, :NQ, :]
    hq = jnp.maximum(_dot(aggq, qW1[...]) + qb1[...][None, :], 0.0)
    pQ[...] = _dot(hq, qW2[...])
    hgq = _dot(qx[...], gW[...])
    hgd = _dot(dx[...], gW[...])
    hg[0:NQ, :] = hgq
    hg[NQ:NG, :] = hgd
    hg[NG:NG_P, :] = jnp.zeros((NG_P - NG, D), jnp.float32)
    a_s = ga_src[...][None, :]
    a_d = ga_dst[...][None, :]
    vq_s = jnp.sum(hgq * a_s, axis=1)
    vd_s = jnp.sum(hgd * a_s, axis=1)
    vq_d = jnp.sum(hgq * a_d, axis=1)
    vd_d = jnp.sum(hgd * a_d, axis=1)
    esq[...] = vq_s
    esd[...] = vd_s
    edq[...] = vq_d
    edd[...] = vd_d
    mes = jnp.maximum(jnp.maximum(jnp.max(vq_s), jnp.max(vd_s)), 0.0)
    med = jnp.maximum(jnp.maximum(jnp.max(vq_d), jnp.max(vd_d)), 0.0)
    mb = mes + med
    m = jnp.maximum(mb, 0.2 * mb)
    mrow[...] = jnp.full((1, D), m, jnp.float32)


def _make_node_out_body(n):
    def body(p, sp, b, out, pool):
        x = p[...] + sp[0, :n, :] + sp[1, :n, :] + b[...][None, :]
        out[...] = x
        pool[...] = jnp.sum(x, axis=0, keepdims=True)
    return body


def _tc_gat_pool_body(outp_p, outden_p, qpool2, dpool2):
    op = outp_p[0] + outp_p[1]
    den = (outden_p[0] + outden_p[1]).reshape(1, NG_P)
    r = 1.0 / (den + 1e-16)
    cols = lax.broadcasted_iota(jnp.int32, (1, NG_P), 1)
    rq = jnp.where(cols < NQ, r, 0.0)
    rd = jnp.where((cols >= NQ) & (cols < NG), r, 0.0)
    qpool2[...] = _dot(rq, op)
    dpool2[...] = _dot(rd, op)


def _tc_mlp_body(qpool, qpool2, dpool, dpool2,
                 l1W, l1b, l2W, l2b, l3W, l3b, l4W, l4b, out):
    pooled = jnp.concatenate(
        [qpool[...], qpool2[...], dpool[...], dpool2[...]], axis=1)
    h = _dot(pooled, l1W[...]) + l1b[...][None, :]
    h = _dot(h, l2W[...]) + l2b[...][None, :]
    h = jnp.maximum(h, 0.0)
    h = _dot(h, l3W[...]) + l3b[...][None, :]
    h = jnp.maximum(h, 0.0)
    h = _dot(h, l4W[...]) + l4b[...][None, :]
    out[...] = jnp.maximum(h, 0.0)


def _pad_edges(edges, n_chunks_per_worker, dummy):
    e = edges.astype(jnp.int32)
    total = NW * n_chunks_per_worker * K
    pad = total - e.shape[1]
    src = jnp.concatenate([e[0], jnp.zeros((pad,), jnp.int32)])
    dst = jnp.concatenate([e[1], jnp.full((pad,), dummy, jnp.int32)])
    return src, dst


def kernel(query_in_feat, data_in_feat, query_edge_list, data_edge_list,
           query2data_edge_list, qW1, qb1, qW2, qb2, dW1, db1, dW2, db2,
           gW, ga_src, ga_dst, l1W, l1b, l2W, l2b, l3W, l3b, l4W, l4b):
    qx = query_in_feat
    dx = data_in_feat
    qsrc, qdst = _pad_edges(query_edge_list, CQ, NQ)
    dsrc, ddst = _pad_edges(data_edge_list, CD, ND)
    gsrc, gdst = _pad_edges(query2data_edge_list, CG, NG)
    z2d = jnp.zeros((NG_P, D), jnp.float32)
    z1d = jnp.zeros((NG_P,), jnp.float32)

    sq1p, sd1p = _segsum_call(qx, dx, qsrc, qdst, dsrc, ddst, z2d)

    pD = pl.pallas_call(
        _tc_data_gin_body,
        grid=(10,),
        in_specs=[
            pl.BlockSpec((1000, D), lambda i: (i, 0)),
            pl.BlockSpec((NC, 1000, D), lambda i: (0, i, 0)),
            pl.BlockSpec((D, 256), lambda i: (0, 0)),
            pl.BlockSpec((256,), lambda i: (0,)),
            pl.BlockSpec((256, D), lambda i: (0, 0)),
        ],
        out_specs=pl.BlockSpec((1000, D), lambda i: (i, 0)),
        out_shape=jax.ShapeDtypeStruct((ND, D), jnp.float32),
    )(dx, sd1p, dW1, db1, dW2)

    pQ, hg, esq, esd, edq, edd, mrow = pl.pallas_call(
        _tc_query_gat_body,
        out_shape=(
            jax.ShapeDtypeStruct((NQ, D), jnp.float32),
            jax.ShapeDtypeStruct((NG_P, D), jnp.float32),
            jax.ShapeDtypeStruct((NQ,), jnp.float32),
            jax.ShapeDtypeStruct((ND,), jnp.float32),
            jax.ShapeDtypeStruct((NQ,), jnp.float32),
            jax.ShapeDtypeStruct((ND,), jnp.float32),
            jax.ShapeDtypeStruct((1, D), jnp.float32),
        ),
    )(qx, dx, sq1p, qW1, qb1, qW2, gW, ga_src, ga_dst)

    zpad = jnp.zeros((NG_P - NG,), jnp.float32)
    es = jnp.concatenate([esq, esd, zpad])
    ed = jnp.concatenate([edq, edd, zpad])
    m16 = mrow[0, :16]

    sq2p, sd2p = _segsum_call(pQ, pD, qsrc, qdst, dsrc, ddst, z2d)

    outp_p, outden_p = _gat_call(hg, gsrc, gdst, es, ed, m16, z2d, z1d)

    query_x, qpool = pl.pallas_call(
        _make_node_out_body(NQ),
        out_shape=(
            jax.ShapeDtypeStruct((NQ, D), jnp.float32),
            jax.ShapeDtypeStruct((1, D), jnp.float32),
        ),
    )(pQ, sq2p, qb2)
    data_x, dpool = pl.pallas_call(
        _make_node_out_body(ND),
        out_shape=(
            jax.ShapeDtypeStruct((ND, D), jnp.float32),
            jax.ShapeDtypeStruct((1, D), jnp.float32),
        ),
    )(pD, sd2p, db2)
    qpool2, dpool2 = pl.pallas_call(
        _tc_gat_pool_body,
        out_shape=(
            jax.ShapeDtypeStruct((1, D), jnp.float32),
            jax.ShapeDtypeStruct((1, D), jnp.float32),
        ),
    )(outp_p, outden_p)

    h4 = pl.pallas_call(
        _tc_mlp_body,
        out_shape=jax.ShapeDtypeStruct((1, 8 * 4674), jnp.float32),
    )(qpool, qpool2, dpool, dpool2, l1W, l1b, l2W, l2b, l3W, l3b, l4W, l4b)

    pred = h4.reshape(8, 4674)
    return (pred, query_x, data_x)
